# baseline (speedup 1.0000x reference)
import functools

import jax
import jax.numpy as jnp
from jax.experimental import pallas as pl
from jax.experimental.pallas import tpu as pltpu

B, CIN, COUT, K = 2, 128, 128, 3
H, W = 91, 180
HP = 96
WP = 192
FB = 91
DH, DW = 5, 9
BC = B * CIN
HL = HP + 4


def _stencil_coeffs(psi):
    ho = jnp.arange(HP)
    hoc = jnp.minimum(ho, H - 1)
    hi = ho[:, None] + jnp.arange(DH)[None, :] - 2
    valid = (hi >= 0) & (hi < H) & (ho < H)[:, None]
    hic = jnp.clip(hi, 0, H - 1)
    wi = (jnp.arange(DW) - 4) % W
    g = psi[:, hoc[:, None, None], hic[:, :, None], wi[None, None, :]]
    g = g * valid[None, :, :, None]
    return g.transpose(1, 0, 2, 3).astype(jnp.float32)


def _dft_tables():
    f = jnp.arange(FB, dtype=jnp.float32)
    w = jnp.arange(W, dtype=jnp.float32)
    ang = 2.0 * jnp.pi * w[:, None] * f[None, :] / W
    fwd = jnp.zeros((W, 256), jnp.float32)
    fwd = fwd.at[:, :FB].set(jnp.cos(ang))
    fwd = fwd.at[:, 128:128 + FB].set(-jnp.sin(ang))
    alpha = jnp.where((f == 0) | (f == FB - 1), 1.0, 2.0)
    inv = jnp.zeros((256, WP), jnp.float32)
    inv = inv.at[:FB, :W].set(alpha[:, None] * jnp.cos(ang.T) / W)
    inv = inv.at[128:128 + FB, :W].set(-alpha[:, None] * jnp.sin(ang.T) / W)
    return fwd.astype(jnp.bfloat16), inv.astype(jnp.bfloat16)


def _psi_hat(ps):
    f = jnp.arange(FB, dtype=jnp.float32)
    d = jnp.arange(DW, dtype=jnp.float32) - 4.0
    angd = 2.0 * jnp.pi * f[:, None] * d[None, :] / W
    are = jnp.einsum('gkhd,fd->gkhf', ps, jnp.cos(angd))
    aim = jnp.einsum('gkhd,fd->gkhf', ps, jnp.sin(angd))
    a = jnp.stack([are, aim], axis=3)
    a = a.reshape(HP, K * DH * 2, FB)
    return jnp.pad(a, ((0, 0), (0, 0), (0, 96 - FB)))


def _dft_body(x_ref, fwd_ref, xhat_ref):
    xhat_ref[...] = jax.lax.dot_general(
        x_ref[...], fwd_ref[...], (((2,), (0,)), ((), ())),
        preferred_element_type=jnp.float32)


def _conv_body(xhat_ref, ahat_ref, w2_ref, inv_ref, out_ref, ys_ref):
    g = pl.program_id(0)

    @pl.when(g == 0)
    def _init():
        ys_ref[...] = jnp.zeros((B, K * CIN, 256), jnp.bfloat16)

    a_all = ahat_ref[g]
    for c in range(BC // 32):
        lo = c * 32
        xre = [xhat_ref[g + dh, lo:lo + 32, 0:96] for dh in range(DH)]
        xim = [xhat_ref[g + dh, lo:lo + 32, 128:224] for dh in range(DH)]
        b, off = divmod(lo, CIN)
        for k in range(K):
            accre = jnp.zeros((32, 96), jnp.float32)
            accim = jnp.zeros((32, 96), jnp.float32)
            for dh in range(DH):
                ar = a_all[(k * DH + dh) * 2][None, :]
                ai = a_all[(k * DH + dh) * 2 + 1][None, :]
                accre = accre + ar * xre[dh] - ai * xim[dh]
                accim = accim + ar * xim[dh] + ai * xre[dh]
            ys_ref[b, k * CIN + off:k * CIN + off + 32, 0:96] = (
                accre.astype(jnp.bfloat16))
            ys_ref[b, k * CIN + off:k * CIN + off + 32, 128:224] = (
                accim.astype(jnp.bfloat16))
    for b in range(B):
        outhat = jax.lax.dot_general(
            w2_ref[...], ys_ref[b], (((1,), (0,)), ((), ())),
            preferred_element_type=jnp.float32)
        out_ref[b, 0] = jax.lax.dot_general(
            outhat.astype(jnp.bfloat16), inv_ref[...], (((1,), (0,)), ((), ())),
            preferred_element_type=jnp.float32)


@functools.partial(jax.jit, static_argnames=())
def kernel(x, weight, bias, psi):
    xh = x.reshape(BC, H, W).transpose(1, 0, 2)
    xh = jnp.pad(xh, ((2, HL - H - 2), (0, 0), (0, 0))).astype(jnp.bfloat16)
    fwd, inv = _dft_tables()
    ahat = _psi_hat(_stencil_coeffs(psi))
    w2 = weight.transpose(0, 2, 1).reshape(COUT, K * CIN).astype(jnp.bfloat16)

    xhat = pl.pallas_call(
        _dft_body,
        grid=(4,),
        in_specs=[
            pl.BlockSpec((HL // 4, BC, W), lambda i: (i, 0, 0)),
            pl.BlockSpec((W, 256), lambda i: (0, 0)),
        ],
        out_specs=pl.BlockSpec((HL // 4, BC, 256), lambda i: (i, 0, 0)),
        out_shape=jax.ShapeDtypeStruct((HL, BC, 256), jnp.float32),
    )(xh, fwd)

    out = pl.pallas_call(
        _conv_body,
        grid=(HP,),
        in_specs=[
            pl.BlockSpec((HL, BC, 256), lambda i: (0, 0, 0)),
            pl.BlockSpec((HP, 2 * K * DH, 96), lambda i: (0, 0, 0)),
            pl.BlockSpec((COUT, K * CIN), lambda i: (0, 0)),
            pl.BlockSpec((256, WP), lambda i: (0, 0)),
        ],
        out_specs=pl.BlockSpec((B, 1, COUT, WP), lambda i: (0, i, 0, 0)),
        out_shape=jax.ShapeDtypeStruct((B, HP, COUT, WP), jnp.float32),
        scratch_shapes=[pltpu.VMEM((B, K * CIN, 256), jnp.bfloat16)],
    )(xhat, ahat, w2, inv)

    out = out.transpose(0, 2, 1, 3)[:, :, :H, :W]
    return out + bias[None, :, None, None]
